# initial kernel scaffold (unmeasured)
import jax
import jax.numpy as jnp
from jax import lax
from jax.experimental import pallas as pl
from jax.experimental.pallas import tpu as pltpu

N_DEV = 4
B_SH = 64
B = 256
D = 2048
H_SH = 4096
KT_IN = 1024
KT_HID = 1024


def kernel(x, Win0, Wout0, Win1, Wout1, Win2, Wout2):
    def body(x_ref, win0, wout0, win1, wout1, win2, wout2, out_ref,
             x_full, agbuf, arbuf, win_t, wout_t,
             ag_send, ag_recv, ar_send, ar_recv, wsem):
        my = lax.axis_index("i")
        left = (my - 1) % N_DEV
        right = (my + 1) % N_DEV

        barrier = pltpu.get_barrier_semaphore()
        for nbr in (left, right):
            pl.semaphore_signal(barrier, inc=1, device_id=(nbr,),
                                device_id_type=pl.DeviceIdType.MESH)
        pl.semaphore_wait(barrier, 2)

        mine = x_ref[...].astype(jnp.bfloat16)
        x_full[pl.ds(my * B_SH, B_SH), :] = mine
        agbuf[0, :, :] = mine
        for h in range(N_DEV - 1):
            s, r = h % 2, (h + 1) % 2
            rdma = pltpu.make_async_remote_copy(
                src_ref=agbuf.at[s], dst_ref=agbuf.at[r],
                send_sem=ag_send.at[s], recv_sem=ag_recv.at[r],
                device_id=(right,), device_id_type=pl.DeviceIdType.MESH)
            rdma.start()
            rdma.wait()
            origin = (my - h - 1) % N_DEV
            x_full[pl.ds(origin * B_SH, B_SH), :] = agbuf[r, :, :]

        layers = ((win0, wout0), (win1, wout1), (win2, wout2))
        for li, (win, wout) in enumerate(layers):
            hacc = None
            for t in range(D // KT_IN):
                cp = pltpu.make_async_copy(
                    win.at[pl.ds(t * KT_IN, KT_IN), :], win_t, wsem)
                cp.start()
                cp.wait()
                part = jnp.dot(
                    x_full[:, pl.ds(t * KT_IN, KT_IN)],
                    win_t[...].astype(jnp.bfloat16),
                    preferred_element_type=jnp.float32)
                hacc = part if hacc is None else hacc + part
            hb = jnp.maximum(hacc, 0.0).astype(jnp.bfloat16)

            pacc = None
            for t in range(H_SH // KT_HID):
                cp = pltpu.make_async_copy(
                    wout.at[pl.ds(t * KT_HID, KT_HID), :], wout_t, wsem)
                cp.start()
                cp.wait()
                p = jnp.dot(
                    hb[:, pl.ds(t * KT_HID, KT_HID)],
                    wout_t[...].astype(jnp.bfloat16),
                    preferred_element_type=jnp.float32)
                pacc = p if pacc is None else pacc + p

            arbuf[0, :, :] = pacc.astype(jnp.bfloat16)
            total = pacc
            for h in range(N_DEV - 1):
                s, r = h % 2, (h + 1) % 2
                rdma = pltpu.make_async_remote_copy(
                    src_ref=arbuf.at[s], dst_ref=arbuf.at[r],
                    send_sem=ar_send.at[s], recv_sem=ar_recv.at[r],
                    device_id=(right,), device_id_type=pl.DeviceIdType.MESH)
                rdma.start()
                rdma.wait()
                total = total + arbuf[r, :, :].astype(jnp.float32)

            if li == len(layers) - 1:
                out_ref[...] = total
            else:
                x_full[...] = total.astype(jnp.bfloat16)

    return pl.pallas_call(
        body,
        out_shape=jax.ShapeDtypeStruct((B, D), jnp.float32),
        in_specs=[pl.BlockSpec(memory_space=pltpu.VMEM)]
        + [pl.BlockSpec(memory_space=pltpu.ANY)] * 6,
        out_specs=pl.BlockSpec(memory_space=pltpu.VMEM),
        scratch_shapes=[
            pltpu.VMEM((B, D), jnp.bfloat16),
            pltpu.VMEM((2, B_SH, D), jnp.bfloat16),
            pltpu.VMEM((2, B, D), jnp.bfloat16),
            pltpu.VMEM((KT_IN, H_SH), jnp.float32),
            pltpu.VMEM((KT_HID, D), jnp.float32),
            pltpu.SemaphoreType.DMA((2,)),
            pltpu.SemaphoreType.DMA((2,)),
            pltpu.SemaphoreType.DMA((2,)),
            pltpu.SemaphoreType.DMA((2,)),
            pltpu.SemaphoreType.DMA,
        ],
        compiler_params=pltpu.CompilerParams(collective_id=0),
    )(x, Win0, Wout0, Win1, Wout1, Win2, Wout2)


# baseline (device time: 243624 ns/iter reference)
import jax
import jax.numpy as jnp
from jax import lax
from jax.experimental import pallas as pl
from jax.experimental.pallas import tpu as pltpu

N_DEV = 4
B_SH = 64
B = 256
D = 2048
H_SH = 4096
KT_IN = 1024
KT_HID = 1024


def kernel(x, Win0, Wout0, Win1, Wout1, Win2, Wout2):
    def body(x_ref, win0, wout0, win1, wout1, win2, wout2, out_ref,
             x_full, agbuf, arbuf, win_t, wout_t,
             ag_send, ag_recv, ar_send, ar_recv, wsem):
        my = lax.axis_index("i")
        left = (my - 1) % N_DEV
        right = (my + 1) % N_DEV

        barrier = pltpu.get_barrier_semaphore()
        for nbr in (left, right):
            pl.semaphore_signal(barrier, inc=1, device_id=(nbr,),
                                device_id_type=pl.DeviceIdType.MESH)
        pl.semaphore_wait(barrier, 2)

        mine = x_ref[...].astype(jnp.bfloat16)
        x_full[pl.ds(my * B_SH, B_SH), :] = mine
        agbuf[0, :, :] = mine
        for h in range(N_DEV - 1):
            s, r = h % 2, (h + 1) % 2
            rdma = pltpu.make_async_remote_copy(
                src_ref=agbuf.at[s], dst_ref=agbuf.at[r],
                send_sem=ag_send.at[s], recv_sem=ag_recv.at[r],
                device_id=(right,), device_id_type=pl.DeviceIdType.MESH)
            rdma.start()
            rdma.wait()
            origin = (my - h - 1) % N_DEV
            x_full[pl.ds(origin * B_SH, B_SH), :] = agbuf[r, :, :]

        layers = ((win0, wout0), (win1, wout1), (win2, wout2))
        for li, (win, wout) in enumerate(layers):
            hacc = None
            for t in range(D // KT_IN):
                cp = pltpu.make_async_copy(
                    win.at[pl.ds(t * KT_IN, KT_IN), :], win_t, wsem)
                cp.start()
                cp.wait()
                part = jnp.dot(
                    x_full[:, pl.ds(t * KT_IN, KT_IN)],
                    win_t[...].astype(jnp.bfloat16),
                    preferred_element_type=jnp.float32)
                hacc = part if hacc is None else hacc + part
            hb = jnp.maximum(hacc, 0.0).astype(jnp.bfloat16)

            pacc = None
            for t in range(H_SH // KT_HID):
                cp = pltpu.make_async_copy(
                    wout.at[pl.ds(t * KT_HID, KT_HID), :], wout_t, wsem)
                cp.start()
                cp.wait()
                p = jnp.dot(
                    hb[:, t * KT_HID:(t + 1) * KT_HID],
                    wout_t[...].astype(jnp.bfloat16),
                    preferred_element_type=jnp.float32)
                pacc = p if pacc is None else pacc + p

            arbuf[0, :, :] = pacc.astype(jnp.bfloat16)
            total = pacc
            for h in range(N_DEV - 1):
                s, r = h % 2, (h + 1) % 2
                rdma = pltpu.make_async_remote_copy(
                    src_ref=arbuf.at[s], dst_ref=arbuf.at[r],
                    send_sem=ar_send.at[s], recv_sem=ar_recv.at[r],
                    device_id=(right,), device_id_type=pl.DeviceIdType.MESH)
                rdma.start()
                rdma.wait()
                total = total + arbuf[r, :, :].astype(jnp.float32)

            if li == len(layers) - 1:
                out_ref[...] = total
            else:
                x_full[...] = total.astype(jnp.bfloat16)

    return pl.pallas_call(
        body,
        out_shape=jax.ShapeDtypeStruct((B, D), jnp.float32),
        in_specs=[pl.BlockSpec(memory_space=pltpu.VMEM)]
        + [pl.BlockSpec(memory_space=pl.ANY)] * 6,
        out_specs=pl.BlockSpec(memory_space=pltpu.VMEM),
        scratch_shapes=[
            pltpu.VMEM((B, D), jnp.bfloat16),
            pltpu.VMEM((2, B_SH, D), jnp.bfloat16),
            pltpu.VMEM((2, B, D), jnp.bfloat16),
            pltpu.VMEM((KT_IN, H_SH), jnp.float32),
            pltpu.VMEM((KT_HID, D), jnp.float32),
            pltpu.SemaphoreType.DMA((2,)),
            pltpu.SemaphoreType.DMA((2,)),
            pltpu.SemaphoreType.DMA((2,)),
            pltpu.SemaphoreType.DMA((2,)),
            pltpu.SemaphoreType.DMA,
        ],
        compiler_params=pltpu.CompilerParams(
            collective_id=0, vmem_limit_bytes=60 * 1024 * 1024),
    )(x, Win0, Wout0, Win1, Wout1, Win2, Wout2)


# device time: 118477 ns/iter; 2.0563x vs baseline; 2.0563x over previous
import jax
import jax.numpy as jnp
from jax import lax
from jax.experimental import pallas as pl
from jax.experimental.pallas import tpu as pltpu

N_DEV = 4
B_SH = 64
B = 256
D = 2048
H_SH = 4096
KT_IN = 512
KT_HID = 1024
N_TI = D // KT_IN
N_TO = H_SH // KT_HID


def kernel(x, Win0, Wout0, Win1, Wout1, Win2, Wout2):
    def body(x_ref, win0, wout0, win1, wout1, win2, wout2, out_ref,
             x_full, pbuf, paccbuf, rsrecv, winbuf, woutbuf,
             winsem, woutsem, rs_send, rs_recv, ag_send, ag_recv):
        my = lax.axis_index("i")
        left = (my - 1) % N_DEV
        right = (my + 1) % N_DEV

        wins = (win0, win1, win2)
        wouts = (wout0, wout1, wout2)

        win_jobs = [(l, t) for l in range(3) for t in range(N_TI)]
        wout_jobs = [(l, t) for l in range(3) for t in range(N_TO)]
        win_descs = {}
        wout_descs = {}

        def start_win(k):
            if k < len(win_jobs):
                l, t = win_jobs[k]
                cp = pltpu.make_async_copy(
                    wins[l].at[pl.ds(t * KT_IN, KT_IN), :],
                    winbuf.at[k % 2], winsem.at[k % 2])
                cp.start()
                win_descs[k] = cp

        def start_wout(k):
            if k < len(wout_jobs):
                l, t = wout_jobs[k]
                cp = pltpu.make_async_copy(
                    wouts[l].at[pl.ds(t * KT_HID, KT_HID), :],
                    woutbuf.at[k % 2], woutsem.at[k % 2])
                cp.start()
                wout_descs[k] = cp

        start_win(0)
        start_win(1)
        start_wout(0)
        start_wout(1)

        barrier = pltpu.get_barrier_semaphore()
        for nbr in (left, right):
            pl.semaphore_signal(barrier, inc=1, device_id=(nbr,),
                                device_id_type=pl.DeviceIdType.MESH)
        pl.semaphore_wait(barrier, 2)

        sl_my = (pl.ds(my * B_SH, B_SH), slice(None))

        def ag_round():
            rdmas = []
            for j in (1, 2, 3):
                dst = (my + j) % N_DEV
                r = pltpu.make_async_remote_copy(
                    src_ref=x_full.at[sl_my], dst_ref=x_full.at[sl_my],
                    send_sem=ag_send.at[j - 1], recv_sem=ag_recv.at[j - 1],
                    device_id=(dst,), device_id_type=pl.DeviceIdType.MESH)
                r.start()
                rdmas.append(r)
            for r in rdmas:
                r.wait_recv()
            for r in rdmas:
                r.wait_send()

        x_full[sl_my] = x_ref[...].astype(jnp.bfloat16)
        ag_round()

        for l in range(3):
            hacc = None
            for t in range(N_TI):
                k = l * N_TI + t
                win_descs[k].wait()
                w = winbuf[k % 2].astype(jnp.bfloat16)
                start_win(k + 2)
                part = jnp.dot(x_full[:, t * KT_IN:(t + 1) * KT_IN], w,
                               preferred_element_type=jnp.float32)
                hacc = part if hacc is None else hacc + part
            hb = jnp.maximum(hacc, 0.0).astype(jnp.bfloat16)

            pacc = None
            for t in range(N_TO):
                k = l * N_TO + t
                wout_descs[k].wait()
                w = woutbuf[k % 2].astype(jnp.bfloat16)
                start_wout(k + 2)
                p = jnp.dot(hb[:, t * KT_HID:(t + 1) * KT_HID], w,
                            preferred_element_type=jnp.float32)
                pacc = p if pacc is None else pacc + p
            paccbuf[...] = pacc
            pbuf[...] = pacc.astype(jnp.bfloat16)

            rdmas = []
            for j in (1, 2, 3):
                dst = (my + j) % N_DEV
                r = pltpu.make_async_remote_copy(
                    src_ref=pbuf.at[pl.ds(dst * B_SH, B_SH), :],
                    dst_ref=rsrecv.at[j - 1],
                    send_sem=rs_send.at[j - 1], recv_sem=rs_recv.at[j - 1],
                    device_id=(dst,), device_id_type=pl.DeviceIdType.MESH)
                r.start()
                rdmas.append(r)
            for r in rdmas:
                r.wait_recv()
            red = paccbuf[sl_my]
            for kk in range(3):
                red = red + rsrecv[kk].astype(jnp.float32)
            for r in rdmas:
                r.wait_send()

            x_full[sl_my] = red.astype(jnp.bfloat16)
            ag_round()

        out_ref[...] = x_full[...].astype(jnp.float32)

    return pl.pallas_call(
        body,
        out_shape=jax.ShapeDtypeStruct((B, D), jnp.float32),
        in_specs=[pl.BlockSpec(memory_space=pltpu.VMEM)]
        + [pl.BlockSpec(memory_space=pl.ANY)] * 6,
        out_specs=pl.BlockSpec(memory_space=pltpu.VMEM),
        scratch_shapes=[
            pltpu.VMEM((B, D), jnp.bfloat16),
            pltpu.VMEM((B, D), jnp.bfloat16),
            pltpu.VMEM((B, D), jnp.float32),
            pltpu.VMEM((3, B_SH, D), jnp.bfloat16),
            pltpu.VMEM((2, KT_IN, H_SH), jnp.float32),
            pltpu.VMEM((2, KT_HID, D), jnp.float32),
            pltpu.SemaphoreType.DMA((2,)),
            pltpu.SemaphoreType.DMA((2,)),
            pltpu.SemaphoreType.DMA((3,)),
            pltpu.SemaphoreType.DMA((3,)),
            pltpu.SemaphoreType.DMA((3,)),
            pltpu.SemaphoreType.DMA((3,)),
        ],
        compiler_params=pltpu.CompilerParams(
            collective_id=0, vmem_limit_bytes=60 * 1024 * 1024),
    )(x, Win0, Wout0, Win1, Wout1, Win2, Wout2)


# device time: 113309 ns/iter; 2.1501x vs baseline; 1.0456x over previous
import jax
import jax.numpy as jnp
from jax import lax
from jax.experimental import pallas as pl
from jax.experimental.pallas import tpu as pltpu

N_DEV = 4
B_SH = 64
B = 256
D = 2048
H_SH = 4096
KT_IN = 512
KT_HID = 1024
N_TI = D // KT_IN
N_TO = H_SH // KT_HID


def kernel(x, Win0, Wout0, Win1, Wout1, Win2, Wout2):
    def body(x_ref, win0, wout0, win1, wout1, win2, wout2, out_ref,
             x_full, pbuf, rsrecv, winbuf, winbf, woutbuf,
             winsem, woutsem, rs_send, rs_recv, ag_send, ag_recv):
        my = lax.axis_index("i")
        left = (my - 1) % N_DEV
        right = (my + 1) % N_DEV

        wins = (win0, win1, win2)
        wouts = (wout0, wout1, wout2)

        win_jobs = [(l, t) for l in range(3) for t in range(N_TI)]
        wout_jobs = [(l, t) for l in range(3) for t in range(N_TO)]
        win_descs = {}
        wout_descs = {}

        def start_win(k):
            if k < len(win_jobs):
                l, t = win_jobs[k]
                cp = pltpu.make_async_copy(
                    wins[l].at[pl.ds(t * KT_IN, KT_IN), :],
                    winbuf.at[k % 2], winsem.at[k % 2])
                cp.start()
                win_descs[k] = cp

        def start_wout(k):
            if k < len(wout_jobs):
                l, t = wout_jobs[k]
                cp = pltpu.make_async_copy(
                    wouts[l].at[pl.ds(t * KT_HID, KT_HID), :],
                    woutbuf.at[k % 2], woutsem.at[k % 2])
                cp.start()
                wout_descs[k] = cp

        def cast_win(k):
            win_descs[k].wait()
            winbf[k % 2] = winbuf[k % 2].astype(jnp.bfloat16)
            start_win(k + 2)

        start_win(0)
        start_win(1)
        start_wout(0)
        start_wout(1)

        barrier = pltpu.get_barrier_semaphore()
        for nbr in (left, right):
            pl.semaphore_signal(barrier, inc=1, device_id=(nbr,),
                                device_id_type=pl.DeviceIdType.MESH)
        pl.semaphore_wait(barrier, 2)

        sl_my = (pl.ds(my * B_SH, B_SH), slice(None))

        def ag_start():
            rdmas = []
            for j in (1, 2, 3):
                dst = (my + j) % N_DEV
                r = pltpu.make_async_remote_copy(
                    src_ref=x_full.at[sl_my], dst_ref=x_full.at[sl_my],
                    send_sem=ag_send.at[j - 1], recv_sem=ag_recv.at[j - 1],
                    device_id=(dst,), device_id_type=pl.DeviceIdType.MESH)
                r.start()
                rdmas.append(r)
            return rdmas

        def ag_finish(rdmas):
            for r in rdmas:
                r.wait_recv()
            for r in rdmas:
                r.wait_send()

        x_full[sl_my] = x_ref[...].astype(jnp.bfloat16)
        ag = ag_start()
        cast_win(0)
        cast_win(1)
        ag_finish(ag)

        for l in range(3):
            hacc = None
            for t in range(N_TI):
                k = l * N_TI + t
                if t >= 2:
                    cast_win(k)
                part = jnp.dot(x_full[:, t * KT_IN:(t + 1) * KT_IN],
                               winbf[k % 2],
                               preferred_element_type=jnp.float32)
                hacc = part if hacc is None else hacc + part
            hb = jnp.maximum(hacc, 0.0).astype(jnp.bfloat16)

            pacc = None
            for t in range(N_TO):
                k = l * N_TO + t
                wout_descs[k].wait()
                w = woutbuf[k % 2].astype(jnp.bfloat16)
                start_wout(k + 2)
                p = jnp.dot(hb[:, t * KT_HID:(t + 1) * KT_HID], w,
                            preferred_element_type=jnp.float32)
                pacc = p if pacc is None else pacc + p
            pbuf[...] = pacc.astype(jnp.bfloat16)

            rdmas = []
            for j in (1, 2, 3):
                dst = (my + j) % N_DEV
                r = pltpu.make_async_remote_copy(
                    src_ref=pbuf.at[pl.ds(dst * B_SH, B_SH), :],
                    dst_ref=rsrecv.at[j - 1],
                    send_sem=rs_send.at[j - 1], recv_sem=rs_recv.at[j - 1],
                    device_id=(dst,), device_id_type=pl.DeviceIdType.MESH)
                r.start()
                rdmas.append(r)
            for r in rdmas:
                r.wait_recv()
            red = pbuf[sl_my].astype(jnp.float32)
            for kk in range(3):
                red = red + rsrecv[kk].astype(jnp.float32)
            for r in rdmas:
                r.wait_send()

            x_full[sl_my] = red.astype(jnp.bfloat16)
            ag = ag_start()
            if l + 1 < 3:
                cast_win((l + 1) * N_TI)
                cast_win((l + 1) * N_TI + 1)
            ag_finish(ag)

        out_ref[...] = x_full[...].astype(jnp.float32)

    return pl.pallas_call(
        body,
        out_shape=jax.ShapeDtypeStruct((B, D), jnp.float32),
        in_specs=[pl.BlockSpec(memory_space=pltpu.VMEM)]
        + [pl.BlockSpec(memory_space=pl.ANY)] * 6,
        out_specs=pl.BlockSpec(memory_space=pltpu.VMEM),
        scratch_shapes=[
            pltpu.VMEM((B, D), jnp.bfloat16),
            pltpu.VMEM((B, D), jnp.bfloat16),
            pltpu.VMEM((3, B_SH, D), jnp.bfloat16),
            pltpu.VMEM((2, KT_IN, H_SH), jnp.float32),
            pltpu.VMEM((2, KT_IN, H_SH), jnp.bfloat16),
            pltpu.VMEM((2, KT_HID, D), jnp.float32),
            pltpu.SemaphoreType.DMA((2,)),
            pltpu.SemaphoreType.DMA((2,)),
            pltpu.SemaphoreType.DMA((3,)),
            pltpu.SemaphoreType.DMA((3,)),
            pltpu.SemaphoreType.DMA((3,)),
            pltpu.SemaphoreType.DMA((3,)),
        ],
        compiler_params=pltpu.CompilerParams(
            collective_id=0, vmem_limit_bytes=60 * 1024 * 1024),
    )(x, Win0, Wout0, Win1, Wout1, Win2, Wout2)


# device time: 102577 ns/iter; 2.3750x vs baseline; 1.1046x over previous
import jax
import jax.numpy as jnp
from jax import lax
from jax.experimental import pallas as pl
from jax.experimental.pallas import tpu as pltpu

N_DEV = 4
B_SH = 64
B = 256
D = 2048
H_SH = 4096
KT_IN = 512
KT_HID = 1024
C_HALF = D // 2
N_TI = D // KT_IN
N_TO = H_SH // KT_HID


def kernel(x, Win0, Wout0, Win1, Wout1, Win2, Wout2):
    def body(x_ref, win0, wout0, win1, wout1, win2, wout2, out_ref,
             x_full, pbuf, rsrecv, winbuf, winbf, woutbuf, woutbf,
             winsem, woutsem, rs_send, rs_recv, ag_send, ag_recv):
        my = lax.axis_index("i")
        left = (my - 1) % N_DEV
        right = (my + 1) % N_DEV

        wins = (win0, win1, win2)
        wouts = (wout0, wout1, wout2)

        win_jobs = [(l, t) for l in range(3) for t in range(N_TI)]
        wout_jobs = [(l, c, t) for l in range(3) for c in range(2)
                     for t in range(N_TO)]
        win_descs = {}
        wout_descs = {}

        def start_win(k):
            if k < len(win_jobs):
                l, t = win_jobs[k]
                cp = pltpu.make_async_copy(
                    wins[l].at[pl.ds(t * KT_IN, KT_IN), :],
                    winbuf.at[k % 2], winsem.at[k % 2])
                cp.start()
                win_descs[k] = cp

        def start_wout(k):
            if k < len(wout_jobs):
                l, c, t = wout_jobs[k]
                cp = pltpu.make_async_copy(
                    wouts[l].at[pl.ds(t * KT_HID, KT_HID),
                                pl.ds(c * C_HALF, C_HALF)],
                    woutbuf.at[k % 2], woutsem.at[k % 2])
                cp.start()
                wout_descs[k] = cp

        def cast_win(k):
            if k < len(win_jobs):
                win_descs[k].wait()
                winbf[k % 2] = winbuf[k % 2].astype(jnp.bfloat16)
                start_win(k + 2)

        def cast_wout(k):
            if k < len(wout_jobs):
                wout_descs[k].wait()
                woutbf[k % 2] = woutbuf[k % 2].astype(jnp.bfloat16)
                start_wout(k + 2)

        start_win(0)
        start_win(1)
        start_wout(0)
        start_wout(1)

        barrier = pltpu.get_barrier_semaphore()
        for nbr in (left, right):
            pl.semaphore_signal(barrier, inc=1, device_id=(nbr,),
                                device_id_type=pl.DeviceIdType.MESH)
        pl.semaphore_wait(barrier, 2)

        row_my = pl.ds(my * B_SH, B_SH)

        def ag_start_half(c):
            sl = (row_my, pl.ds(c * C_HALF, C_HALF))
            rdmas = []
            for j in (1, 2, 3):
                dst = (my + j) % N_DEV
                s = (j - 1) * 2 + c
                r = pltpu.make_async_remote_copy(
                    src_ref=x_full.at[sl], dst_ref=x_full.at[sl],
                    send_sem=ag_send.at[s], recv_sem=ag_recv.at[s],
                    device_id=(dst,), device_id_type=pl.DeviceIdType.MESH)
                r.start()
                rdmas.append(r)
            return rdmas

        def rs_start_half(c):
            rdmas = []
            for j in (1, 2, 3):
                dst = (my + j) % N_DEV
                s = (j - 1) * 2 + c
                r = pltpu.make_async_remote_copy(
                    src_ref=pbuf.at[pl.ds(dst * B_SH, B_SH),
                                    pl.ds(c * C_HALF, C_HALF)],
                    dst_ref=rsrecv.at[j - 1, slice(None),
                                      pl.ds(c * C_HALF, C_HALF)],
                    send_sem=rs_send.at[s], recv_sem=rs_recv.at[s],
                    device_id=(dst,), device_id_type=pl.DeviceIdType.MESH)
                r.start()
                rdmas.append(r)
            return rdmas

        x_full[row_my, :] = x_ref[...].astype(jnp.bfloat16)
        ag = ag_start_half(0) + ag_start_half(1)
        cast_win(0)
        cast_win(1)
        cast_wout(0)
        cast_wout(1)
        for r in ag:
            r.wait_recv()
        for r in ag:
            r.wait_send()

        for l in range(3):
            hacc = None
            for t in range(N_TI):
                k = l * N_TI + t
                part = jnp.dot(x_full[:, t * KT_IN:(t + 1) * KT_IN],
                               winbf[k % 2],
                               preferred_element_type=jnp.float32)
                hacc = part if hacc is None else hacc + part
                if t < N_TI - 2:
                    cast_win(k + 2)
            hb = jnp.maximum(hacc, 0.0).astype(jnp.bfloat16)

            kb = l * 2 * N_TO
            rs_all = []
            for c in range(2):
                pacc = None
                for t in range(N_TO):
                    k = kb + c * N_TO + t
                    p = jnp.dot(hb[:, t * KT_HID:(t + 1) * KT_HID],
                                woutbf[k % 2],
                                preferred_element_type=jnp.float32)
                    pacc = p if pacc is None else pacc + p
                    if not (c == 1 and t >= N_TO - 2):
                        cast_wout(k + 2)
                pbuf[:, c * C_HALF:(c + 1) * C_HALF] = pacc.astype(jnp.bfloat16)
                rs_all.append(rs_start_half(c))

            ag_all = []
            for c in range(2):
                for r in rs_all[c]:
                    r.wait_recv()
                csl = pl.ds(c * C_HALF, C_HALF)
                red = pbuf[row_my, csl].astype(jnp.float32)
                for kk in range(3):
                    red = red + rsrecv[kk, :, c * C_HALF:(c + 1) * C_HALF
                                       ].astype(jnp.float32)
                x_full[row_my, csl] = red.astype(jnp.bfloat16)
                ag_all += ag_start_half(c)

            if l + 1 < 3:
                cast_win((l + 1) * N_TI)
                cast_win((l + 1) * N_TI + 1)
                cast_wout((l + 1) * 2 * N_TO)
                cast_wout((l + 1) * 2 * N_TO + 1)
            for r in ag_all:
                r.wait_recv()
            for r in ag_all:
                r.wait_send()
            for rl in rs_all:
                for r in rl:
                    r.wait_send()

        out_ref[...] = x_full[...].astype(jnp.float32)

    return pl.pallas_call(
        body,
        out_shape=jax.ShapeDtypeStruct((B, D), jnp.float32),
        in_specs=[pl.BlockSpec(memory_space=pltpu.VMEM)]
        + [pl.BlockSpec(memory_space=pl.ANY)] * 6,
        out_specs=pl.BlockSpec(memory_space=pltpu.VMEM),
        scratch_shapes=[
            pltpu.VMEM((B, D), jnp.bfloat16),
            pltpu.VMEM((B, D), jnp.bfloat16),
            pltpu.VMEM((3, B_SH, D), jnp.bfloat16),
            pltpu.VMEM((2, KT_IN, H_SH), jnp.float32),
            pltpu.VMEM((2, KT_IN, H_SH), jnp.bfloat16),
            pltpu.VMEM((2, KT_HID, C_HALF), jnp.float32),
            pltpu.VMEM((2, KT_HID, C_HALF), jnp.bfloat16),
            pltpu.SemaphoreType.DMA((2,)),
            pltpu.SemaphoreType.DMA((2,)),
            pltpu.SemaphoreType.DMA((6,)),
            pltpu.SemaphoreType.DMA((6,)),
            pltpu.SemaphoreType.DMA((6,)),
            pltpu.SemaphoreType.DMA((6,)),
        ],
        compiler_params=pltpu.CompilerParams(
            collective_id=0, vmem_limit_bytes=60 * 1024 * 1024),
    )(x, Win0, Wout0, Win1, Wout1, Win2, Wout2)


# device time: 102165 ns/iter; 2.3846x vs baseline; 1.0040x over previous
import jax
import jax.numpy as jnp
from jax import lax
from jax.experimental import pallas as pl
from jax.experimental.pallas import tpu as pltpu

N_DEV = 4
B_SH = 64
B = 256
D = 2048
H_SH = 4096
KT_IN = 512
KT_HID = 1024
C_HALF = D // 2
N_TI = D // KT_IN
N_TO = H_SH // KT_HID


def kernel(x, Win0, Wout0, Win1, Wout1, Win2, Wout2):
    def body(x_ref, win0, wout0, win1, wout1, win2, wout2, out_ref,
             x_full, pbuf, rsrecv, hb, winbuf, winbf, woutbuf, woutbf,
             winsem, woutsem, rs_send, rs_recv, ag_send, ag_recv):
        my = lax.axis_index("i")
        left = (my - 1) % N_DEV
        right = (my + 1) % N_DEV

        wins = (win0, win1, win2)
        wouts = (wout0, wout1, wout2)

        win_jobs = [(l, t) for l in range(3) for t in range(N_TI)]
        wout_jobs = [(l, c, t) for l in range(3) for c in range(2)
                     for t in range(N_TO)]
        win_descs = {}
        wout_descs = {}

        def start_win(k):
            if k < len(win_jobs):
                l, t = win_jobs[k]
                cp = pltpu.make_async_copy(
                    wins[l].at[pl.ds(t * KT_IN, KT_IN), :],
                    winbuf.at[k % 2], winsem.at[k % 2])
                cp.start()
                win_descs[k] = cp

        def start_wout(k):
            if k < len(wout_jobs):
                l, c, t = wout_jobs[k]
                cp = pltpu.make_async_copy(
                    wouts[l].at[pl.ds(t * KT_HID, KT_HID),
                                pl.ds(c * C_HALF, C_HALF)],
                    woutbuf.at[k % 2], woutsem.at[k % 2])
                cp.start()
                wout_descs[k] = cp

        def cast_win(k):
            if k < len(win_jobs):
                win_descs[k].wait()
                winbf[k % 4] = winbuf[k % 2].astype(jnp.bfloat16)
                start_win(k + 2)

        def cast_wout(k):
            if k < len(wout_jobs):
                wout_descs[k].wait()
                woutbf[k % 2] = woutbuf[k % 2].astype(jnp.bfloat16)
                start_wout(k + 2)

        start_win(0)
        start_win(1)
        start_wout(0)
        start_wout(1)

        barrier = pltpu.get_barrier_semaphore()
        for nbr in (left, right):
            pl.semaphore_signal(barrier, inc=1, device_id=(nbr,),
                                device_id_type=pl.DeviceIdType.MESH)
        pl.semaphore_wait(barrier, 2)

        def ag_start_half(c):
            csl = pl.ds(c * C_HALF, C_HALF)
            rdmas = []
            for j in (1, 2, 3):
                dst = (my + j) % N_DEV
                s = (j - 1) * 2 + c
                r = pltpu.make_async_remote_copy(
                    src_ref=x_full.at[pl.ds(0, B_SH), csl],
                    dst_ref=x_full.at[pl.ds((N_DEV - j) * B_SH, B_SH), csl],
                    send_sem=ag_send.at[s], recv_sem=ag_recv.at[s],
                    device_id=(dst,), device_id_type=pl.DeviceIdType.MESH)
                r.start()
                rdmas.append(r)
            return rdmas

        def rs_start_half(c):
            csl = pl.ds(c * C_HALF, C_HALF)
            rdmas = []
            for j in (1, 2, 3):
                dst = (my + j) % N_DEV
                s = (j - 1) * 2 + c
                r = pltpu.make_async_remote_copy(
                    src_ref=pbuf.at[pl.ds(j * B_SH, B_SH), csl],
                    dst_ref=rsrecv.at[j - 1, slice(None), csl],
                    send_sem=rs_send.at[s], recv_sem=rs_recv.at[s],
                    device_id=(dst,), device_id_type=pl.DeviceIdType.MESH)
                r.start()
                rdmas.append(r)
            return rdmas

        def block0_dots(l, ts, hacc0):
            for t in ts:
                part = jnp.dot(x_full[0:B_SH, t * KT_IN:(t + 1) * KT_IN],
                               winbf[(l * N_TI + t) % 4],
                               preferred_element_type=jnp.float32)
                hacc0 = part if hacc0 is None else hacc0 + part
            return hacc0

        x_full[0:B_SH, :] = x_ref[...].astype(jnp.bfloat16)
        ag = ag_start_half(0) + ag_start_half(1)
        cast_win(0)
        cast_win(1)
        cast_wout(0)
        cast_wout(1)
        hacc0 = block0_dots(0, (0, 1), None)
        cast_win(2)
        cast_win(3)
        hacc0 = block0_dots(0, (2, 3), hacc0)
        hb[0:B_SH, :] = jnp.maximum(hacc0, 0.0).astype(jnp.bfloat16)
        for r in ag:
            r.wait_recv()
        for r in ag:
            r.wait_send()

        for l in range(3):
            haccr = None
            for t in range(N_TI):
                part = jnp.dot(
                    x_full[B_SH:B, t * KT_IN:(t + 1) * KT_IN],
                    winbf[(l * N_TI + t) % 4],
                    preferred_element_type=jnp.float32)
                haccr = part if haccr is None else haccr + part
            hb[B_SH:B, :] = jnp.maximum(haccr, 0.0).astype(jnp.bfloat16)

            kb = l * 2 * N_TO
            rs_all = []
            for c in range(2):
                pacc = None
                for t in range(N_TO):
                    k = kb + c * N_TO + t
                    p = jnp.dot(hb[:, t * KT_HID:(t + 1) * KT_HID],
                                woutbf[k % 2],
                                preferred_element_type=jnp.float32)
                    pacc = p if pacc is None else pacc + p
                    if not (c == 1 and t >= N_TO - 2):
                        cast_wout(k + 2)
                pbuf[:, c * C_HALF:(c + 1) * C_HALF] = pacc.astype(jnp.bfloat16)
                rs_all.append(rs_start_half(c))

            last = l + 1 == 3
            nw = (l + 1) * N_TI
            if not last:
                cast_win(nw)
                cast_win(nw + 1)
            for r in rs_all[0]:
                r.wait_recv()
            c0 = pl.ds(0, C_HALF)
            red = pbuf[0:B_SH, 0:C_HALF].astype(jnp.float32)
            for kk in range(3):
                red = red + rsrecv[kk, :, 0:C_HALF].astype(jnp.float32)
            x_full[0:B_SH, 0:C_HALF] = red.astype(jnp.bfloat16)
            ag_all = ag_start_half(0)
            hacc0 = None
            if not last:
                hacc0 = block0_dots(l + 1, (0, 1), None)
                cast_wout((l + 1) * 2 * N_TO)
                cast_wout((l + 1) * 2 * N_TO + 1)
            for r in rs_all[1]:
                r.wait_recv()
            red = pbuf[0:B_SH, C_HALF:D].astype(jnp.float32)
            for kk in range(3):
                red = red + rsrecv[kk, :, C_HALF:D].astype(jnp.float32)
            x_full[0:B_SH, C_HALF:D] = red.astype(jnp.bfloat16)
            ag_all += ag_start_half(1)
            if not last:
                cast_win(nw + 2)
                cast_win(nw + 3)
                hacc0 = block0_dots(l + 1, (2, 3), hacc0)
                hb[0:B_SH, :] = jnp.maximum(hacc0, 0.0).astype(jnp.bfloat16)
            for r in ag_all:
                r.wait_recv()
            for r in ag_all:
                r.wait_send()
            for rl in rs_all:
                for r in rl:
                    r.wait_send()

        for j in range(N_DEV):
            gb = (my + j) % N_DEV
            out_ref[pl.ds(gb * B_SH, B_SH), :] = (
                x_full[j * B_SH:(j + 1) * B_SH, :].astype(jnp.float32))

    return pl.pallas_call(
        body,
        out_shape=jax.ShapeDtypeStruct((B, D), jnp.float32),
        in_specs=[pl.BlockSpec(memory_space=pltpu.VMEM)]
        + [pl.BlockSpec(memory_space=pl.ANY)] * 6,
        out_specs=pl.BlockSpec(memory_space=pltpu.VMEM),
        scratch_shapes=[
            pltpu.VMEM((B, D), jnp.bfloat16),
            pltpu.VMEM((B, D), jnp.bfloat16),
            pltpu.VMEM((3, B_SH, D), jnp.bfloat16),
            pltpu.VMEM((B, H_SH), jnp.bfloat16),
            pltpu.VMEM((2, KT_IN, H_SH), jnp.float32),
            pltpu.VMEM((4, KT_IN, H_SH), jnp.bfloat16),
            pltpu.VMEM((2, KT_HID, C_HALF), jnp.float32),
            pltpu.VMEM((2, KT_HID, C_HALF), jnp.bfloat16),
            pltpu.SemaphoreType.DMA((2,)),
            pltpu.SemaphoreType.DMA((2,)),
            pltpu.SemaphoreType.DMA((6,)),
            pltpu.SemaphoreType.DMA((6,)),
            pltpu.SemaphoreType.DMA((6,)),
            pltpu.SemaphoreType.DMA((6,)),
        ],
        compiler_params=pltpu.CompilerParams(
            collective_id=0, vmem_limit_bytes=62 * 1024 * 1024),
    )(x, Win0, Wout0, Win1, Wout1, Win2, Wout2)
